# baseline (speedup 1.0000x reference)
import functools

import jax
import jax.numpy as jnp
from jax import lax
from jax.experimental import pallas as pl
from jax.experimental.pallas import tpu as pltpu
from jax.experimental.pallas import tpu_sc as plsc

_SB = 4
_W = 128
_MC = 256
_NBUF = 3


def _tc_part_argmax(blk):
    m = blk.shape[0]
    mx = jnp.max(blk, axis=0)
    iota = lax.broadcasted_iota(jnp.int32, blk.shape, 0)
    idx = jnp.min(jnp.where(blk == mx[None, :], iota, m), axis=0)
    return mx, idx


def _tc_body(x1_ref, x2_ref, o_ref):
    m1 = x1_ref.shape[1]
    mx1, idx1 = _tc_part_argmax(x1_ref[0])
    mx2, idx2 = _tc_part_argmax(x2_ref[0])
    first_low = mx1 >= mx2
    o_ref[0, 0] = jnp.where(first_low, idx1, idx2 + m1)


def _tc_argmax(x, bt):
    B, M, N = x.shape
    MH = M // 2
    out = pl.pallas_call(
        _tc_body,
        grid=(bt,),
        in_specs=[
            pl.BlockSpec((1, MH, N), lambda b: (b, 0, 0)),
            pl.BlockSpec((1, MH, N), lambda b: (b, 1, 0)),
        ],
        out_specs=pl.BlockSpec((1, 1, N), lambda b: (b, 0, 0)),
        out_shape=jax.ShapeDtypeStruct((bt, 1, N), jnp.int32),
    )(x, x)
    return out.reshape(bt, N)


def _sc_argmax(x, b0):
    B, M, N = x.shape
    wpb = N // _W
    nchunks = M // _MC
    ngroups = _W // 16
    mesh = plsc.VectorSubcoreMesh(core_axis_name="c", subcore_axis_name="s")

    @functools.partial(
        pl.kernel,
        out_type=jax.ShapeDtypeStruct((_SB, N), jnp.int32),
        mesh=mesh,
        scratch_types=[
            *[pltpu.VMEM((_MC, _W), jnp.float32) for _ in range(_NBUF)],
            pltpu.VMEM((_W,), jnp.int32),
            *[pltpu.SemaphoreType.DMA for _ in range(_NBUF)],
        ],
    )
    def sc_kernel(x_hbm, out_hbm, buf0, buf1, buf2, idx_v, sem0, sem1, sem2):
        bufs = (buf0, buf1, buf2)
        sems = (sem0, sem1, sem2)
        wid = lax.axis_index("c") * 16 + lax.axis_index("s")
        b = b0 + wid // wpb
        n0 = (wid % wpb) * _W

        def start(c):
            return pltpu.async_copy(
                x_hbm.at[b, pl.ds(c * _MC, _MC), pl.ds(n0, _W)],
                bufs[c % _NBUF],
                sems[c % _NBUF],
            )

        state = [
            (jnp.full((16,), -jnp.inf, jnp.float32), jnp.zeros((16,), jnp.int32))
            for _ in range(ngroups)
        ]

        descs = [None] * nchunks
        for c in range(min(_NBUF, nchunks)):
            descs[c] = start(c)
        for c in range(nchunks):
            descs[c].wait()
            buf = bufs[c % _NBUF]
            m_base = c * _MC
            for g in range(ngroups):
                sl = pl.ds(g * 16, 16)
                cm, ci = state[g]

                def step(i, st, buf=buf, sl=sl, m_base=m_base):
                    scm, sci = st
                    v = buf[i, sl]
                    gt = v > scm
                    scm = jnp.where(gt, v, scm)
                    mvec = jnp.broadcast_to(m_base + i, (16,)).astype(jnp.int32)
                    sci = jnp.where(gt, mvec, sci)
                    return scm, sci

                state[g] = lax.fori_loop(0, _MC, step, (cm, ci))
            if c + _NBUF < nchunks:
                descs[c + _NBUF] = start(c + _NBUF)

        for g in range(ngroups):
            idx_v[pl.ds(g * 16, 16)] = state[g][1]
        pltpu.sync_copy(idx_v, out_hbm.at[b - b0, pl.ds(n0, _W)])

    return sc_kernel(x)


def kernel(x):
    B, M, N = x.shape
    bt = B - _SB
    out_tc = _tc_argmax(x, bt)
    out_sc = _sc_argmax(x, bt)
    return jnp.concatenate([out_tc, out_sc], axis=0)


# speedup vs baseline: 1.7894x; 1.7894x over previous
import functools

import jax
import jax.numpy as jnp
from jax import lax
from jax.experimental import pallas as pl
from jax.experimental.pallas import tpu as pltpu
from jax.experimental.pallas import tpu_sc as plsc

_SB = 4
_W = 128
_MC = 256
_NBUF = 2
_U = 8


def _tc_part_argmax(blk):
    m = blk.shape[0]
    mx = jnp.max(blk, axis=0)
    iota = lax.broadcasted_iota(jnp.int32, blk.shape, 0)
    idx = jnp.min(jnp.where(blk == mx[None, :], iota, m), axis=0)
    return mx, idx


def _tc_body(x1_ref, x2_ref, o_ref):
    m1 = x1_ref.shape[1]
    mx1, idx1 = _tc_part_argmax(x1_ref[0])
    mx2, idx2 = _tc_part_argmax(x2_ref[0])
    first_low = mx1 >= mx2
    o_ref[0, 0] = jnp.where(first_low, idx1, idx2 + m1)


def _tc_argmax(x, bt):
    B, M, N = x.shape
    MH = M // 2
    out = pl.pallas_call(
        _tc_body,
        grid=(bt,),
        in_specs=[
            pl.BlockSpec((1, MH, N), lambda b: (b, 0, 0)),
            pl.BlockSpec((1, MH, N), lambda b: (b, 1, 0)),
        ],
        out_specs=pl.BlockSpec((1, 1, N), lambda b: (b, 0, 0)),
        out_shape=jax.ShapeDtypeStruct((bt, 1, N), jnp.int32),
    )(x, x)
    return out.reshape(bt, N)


def _sc_argmax(x, b0):
    B, M, N = x.shape
    wpb = N // _W
    nchunks = M // _MC
    ngroups = _W // 16
    mesh = plsc.VectorSubcoreMesh(core_axis_name="c", subcore_axis_name="s")

    @functools.partial(
        pl.kernel,
        out_type=jax.ShapeDtypeStruct((_SB, N), jnp.int32),
        mesh=mesh,
        scratch_types=[
            *[pltpu.VMEM((_MC, _W), jnp.float32) for _ in range(_NBUF)],
            pltpu.VMEM((_W,), jnp.int32),
            *[pltpu.SemaphoreType.DMA for _ in range(_NBUF)],
        ],
    )
    def sc_kernel(x_hbm, out_hbm, buf0, buf1, idx_v, sem0, sem1):
        bufs = (buf0, buf1)
        sems = (sem0, sem1)
        wid = lax.axis_index("c") * 16 + lax.axis_index("s")
        b = b0 + wid // wpb
        n0 = (wid % wpb) * _W

        def start(c, j):
            pltpu.async_copy(
                x_hbm.at[b, pl.ds(c * _MC, _MC), pl.ds(n0, _W)],
                bufs[j],
                sems[j],
            )

        def compute_chunk(buf, m_base, state):
            new_state = []
            for g in range(ngroups):
                sl = pl.ds(g * 16, 16)
                cm, ci = state[g]
                mb = jnp.broadcast_to(m_base, (16,)).astype(jnp.int32)

                def step(i, st, buf=buf, sl=sl, mb=mb):
                    scm, sci = st
                    r0 = i * _U
                    for d in range(_U):
                        v = buf[r0 + d, sl]
                        gt = v > scm
                        scm = jnp.where(gt, v, scm)
                        sci = jnp.where(gt, mb + (r0 + d), sci)
                    return scm, sci

                new_state.append(lax.fori_loop(0, _MC // _U, step, (cm, ci)))
            return tuple(new_state)

        for j in range(_NBUF):
            start(j, j)

        state0 = tuple(
            (jnp.full((16,), -jnp.inf, jnp.float32), jnp.zeros((16,), jnp.int32))
            for _ in range(ngroups)
        )

        def outer(k, state):
            for j in range(_NBUF):
                c = k * _NBUF + j
                pltpu.make_async_copy(
                    x_hbm.at[b, pl.ds(0, _MC), pl.ds(n0, _W)], bufs[j], sems[j]
                ).wait()
                state = compute_chunk(bufs[j], c * _MC, state)

                @pl.when(c + _NBUF < nchunks)
                def _prefetch(c=c, j=j):
                    start(c + _NBUF, j)

            return state

        state = lax.fori_loop(0, nchunks // _NBUF, outer, state0)

        for g in range(ngroups):
            idx_v[pl.ds(g * 16, 16)] = state[g][1]
        pltpu.sync_copy(idx_v, out_hbm.at[b - b0, pl.ds(n0, _W)])

    return sc_kernel(x)


def kernel(x):
    B, M, N = x.shape
    bt = B - _SB
    out_tc = _tc_argmax(x, bt)
    out_sc = _sc_argmax(x, bt)
    return jnp.concatenate([out_tc, out_sc], axis=0)
